# baseline (speedup 1.0000x reference)
import functools

import jax
import jax.numpy as jnp
from jax import lax
from jax.experimental import pallas as pl
from jax.experimental.pallas import tpu as pltpu
from jax.experimental.pallas import tpu_sc as plsc

B, S, D, E = 2, 2048, 1024, 8
S_BLK = 2048
NS = S // S_BLK
RC = 64
NRC = S_BLK // RC
NK = D // 128
EPS2 = 1e-24

RW = 4 * D
T_ROWS = E * B * S * D // RW
OUT_ROWS = B * S * D // RW
ROWS_PER_B = OUT_ROWS // B
NW = 32
W_ROWS = OUT_ROWS // NW
GR = 16
NROUND = W_ROWS // GR


def _sims_kernel(s_ref, t_ref, idxr_ref, acc_ref, rs_ref):
    s = pl.program_id(0)
    e = pl.program_id(1)
    for b in range(B):
        @pl.when(e == 0)
        def _():
            for rc in range(NRC):
                r0 = rc * RC
                sn_acc = jnp.zeros((RC, 128), jnp.float32)
                for k in range(NK):
                    sfk = s_ref[b, r0:r0 + RC, k * 128:(k + 1) * 128]
                    sn_acc += sfk * sfk
                sn2 = jnp.sum(sn_acc, axis=1, keepdims=True)
                rs_ref[b, r0:r0 + RC, :] = lax.rsqrt(jnp.maximum(sn2, EPS2))

        part = jnp.zeros((1, 1), jnp.float32)
        for rc in range(NRC):
            r0 = rc * RC
            dot_acc = jnp.zeros((RC, 128), jnp.float32)
            tn_acc = jnp.zeros((RC, 128), jnp.float32)
            for k in range(NK):
                sfk = s_ref[b, r0:r0 + RC, k * 128:(k + 1) * 128]
                tfk = t_ref[0, b, r0:r0 + RC, k * 128:(k + 1) * 128]
                dot_acc += sfk * tfk
                tn_acc += tfk * tfk
            dot = jnp.sum(dot_acc, axis=1, keepdims=True)
            tn2 = jnp.sum(tn_acc, axis=1, keepdims=True)
            rt = lax.rsqrt(jnp.maximum(tn2, EPS2))
            w = dot * rt * rs_ref[b, r0:r0 + RC, :]
            part += jnp.sum(w, axis=0, keepdims=True)
        prev = acc_ref[b, pl.ds(e, 1), :]
        acc_ref[b, pl.ds(e, 1), :] = jnp.where(s == 0, part, prev + part)

    @pl.when((s == NS - 1) & (e == E - 1))
    def _():
        i0 = jnp.argmax(acc_ref[0][:, 0], axis=0).astype(jnp.int32)
        i1 = jnp.argmax(acc_ref[1][:, 0], axis=0).astype(jnp.int32)
        j = (lax.broadcasted_iota(jnp.int32, (8, 128), 0) * 128
             + lax.broadcasted_iota(jnp.int32, (8, 128), 1))
        idxb = jnp.where(j >= ROWS_PER_B, i1, i0)
        idxr_ref[...] = idxb * OUT_ROWS + j


_SC_MESH = plsc.VectorSubcoreMesh(core_axis_name="c", subcore_axis_name="s")


@functools.partial(
    pl.kernel,
    mesh=_SC_MESH,
    out_type=jax.ShapeDtypeStruct((OUT_ROWS, RW), jnp.float32),
    scratch_types=[
        pltpu.VMEM((W_ROWS,), jnp.int32),
        pltpu.VMEM((GR, RW), jnp.float32),
        pltpu.SemaphoreType.DMA,
    ],
)
def _sc_dispatch(t_rows, idx_rows, out_rows, idx_v, buf, sem):
    wid = lax.axis_index("s") * 2 + lax.axis_index("c")
    base = wid * W_ROWS
    pltpu.sync_copy(idx_rows.at[pl.ds(base, W_ROWS)], idx_v)
    for r in range(NROUND):
        pltpu.async_copy(t_rows.at[idx_v.at[pl.ds(r * GR, GR)]], buf, sem).wait()
        pltpu.sync_copy(buf, out_rows.at[pl.ds(base + r * GR, GR)])


@jax.jit
def kernel(student_features, teacher_features):
    idx_rows = pl.pallas_call(
        _sims_kernel,
        grid=(NS, E),
        in_specs=[
            pl.BlockSpec((B, S_BLK, D), lambda s, e: (0, s, 0)),
            pl.BlockSpec((1, B, S_BLK, D), lambda s, e: (e, 0, s, 0)),
        ],
        out_specs=pl.BlockSpec((8, 128), lambda s, e: (0, 0)),
        out_shape=jax.ShapeDtypeStruct((8, 128), jnp.int32),
        scratch_shapes=[
            pltpu.VMEM((B, E, 1), jnp.float32),
            pltpu.VMEM((B, S_BLK, 1), jnp.float32),
        ],
        compiler_params=pltpu.CompilerParams(
            dimension_semantics=("arbitrary", "arbitrary"),
        ),
    )(student_features, teacher_features)

    t_rows = teacher_features.reshape(T_ROWS, RW)
    out = _sc_dispatch(t_rows, idx_rows.reshape(OUT_ROWS))
    return out.reshape(B, S, D)


# speedup vs baseline: 3.1887x; 3.1887x over previous
import functools

import jax
import jax.numpy as jnp
from jax import lax
from jax.experimental import pallas as pl
from jax.experimental.pallas import tpu as pltpu
from jax.experimental.pallas import tpu_sc as plsc

B, S, D, E = 2, 2048, 1024, 8
S_BLK = 2048
NS = S // S_BLK
RC = 64
NRC = S_BLK // RC
NK = D // 128
EPS2 = 1e-24

NW = 32
W_S = S // 16
SLAB = 64
NROUND = W_S // SLAB


def _sims_kernel(s_ref, t_ref, idxr_ref, acc_ref, rs_ref):
    s = pl.program_id(0)
    e = pl.program_id(1)
    for b in range(B):
        @pl.when(e == 0)
        def _():
            for rc in range(NRC):
                r0 = rc * RC
                sn_acc = jnp.zeros((RC, 128), jnp.float32)
                for k in range(NK):
                    sfk = s_ref[b, r0:r0 + RC, k * 128:(k + 1) * 128]
                    sn_acc += sfk * sfk
                sn2 = jnp.sum(sn_acc, axis=1, keepdims=True)
                rs_ref[b, r0:r0 + RC, :] = lax.rsqrt(jnp.maximum(sn2, EPS2))

        part = jnp.zeros((1, 1), jnp.float32)
        for rc in range(NRC):
            r0 = rc * RC
            dot_acc = jnp.zeros((RC, 128), jnp.float32)
            tn_acc = jnp.zeros((RC, 128), jnp.float32)
            for k in range(NK):
                sfk = s_ref[b, r0:r0 + RC, k * 128:(k + 1) * 128]
                tfk = t_ref[0, b, r0:r0 + RC, k * 128:(k + 1) * 128]
                dot_acc += sfk * tfk
                tn_acc += tfk * tfk
            dot = jnp.sum(dot_acc, axis=1, keepdims=True)
            tn2 = jnp.sum(tn_acc, axis=1, keepdims=True)
            rt = lax.rsqrt(jnp.maximum(tn2, EPS2))
            w = dot * rt * rs_ref[b, r0:r0 + RC, :]
            part += jnp.sum(w, axis=0, keepdims=True)
        prev = acc_ref[b, pl.ds(e, 1), :]
        acc_ref[b, pl.ds(e, 1), :] = jnp.where(s == 0, part, prev + part)

    @pl.when((s == NS - 1) & (e == E - 1))
    def _():
        i0 = jnp.argmax(acc_ref[0][:, 0], axis=0).astype(jnp.int32)
        i1 = jnp.argmax(acc_ref[1][:, 0], axis=0).astype(jnp.int32)
        lane = lax.broadcasted_iota(jnp.int32, (8, 128), 1)
        idxr_ref[...] = jnp.where(lane == 0, i0, jnp.where(lane == 1, i1, 0))


_SC_MESH = plsc.VectorSubcoreMesh(core_axis_name="c", subcore_axis_name="s")


@functools.partial(
    pl.kernel,
    mesh=_SC_MESH,
    out_type=jax.ShapeDtypeStruct((B, S, D), jnp.float32),
    scratch_types=[
        pltpu.VMEM((16,), jnp.int32),
        pltpu.VMEM((SLAB, D), jnp.float32),
        pltpu.SemaphoreType.DMA,
    ],
)
def _sc_dispatch(teacher, idx16, out, idx_v, buf, sem):
    wid = lax.axis_index("s") * 2 + lax.axis_index("c")
    b = wid // 16
    s0 = (wid % 16) * W_S
    pltpu.sync_copy(idx16.at[pl.ds(0, 16)], idx_v)
    iv = idx_v[...]
    ib = jnp.where(b == 0, iv[0], iv[1])
    for r in range(NROUND):
        off = s0 + r * SLAB
        pltpu.async_copy(teacher.at[ib, b, pl.ds(off, SLAB)], buf, sem).wait()
        pltpu.sync_copy(buf, out.at[b, pl.ds(off, SLAB)])


@jax.jit
def kernel(student_features, teacher_features):
    idx_rows = pl.pallas_call(
        _sims_kernel,
        grid=(NS, E),
        in_specs=[
            pl.BlockSpec((B, S_BLK, D), lambda s, e: (0, s, 0)),
            pl.BlockSpec((1, B, S_BLK, D), lambda s, e: (e, 0, s, 0)),
        ],
        out_specs=pl.BlockSpec((8, 128), lambda s, e: (0, 0)),
        out_shape=jax.ShapeDtypeStruct((8, 128), jnp.int32),
        scratch_shapes=[
            pltpu.VMEM((B, E, 1), jnp.float32),
            pltpu.VMEM((B, S_BLK, 1), jnp.float32),
        ],
        compiler_params=pltpu.CompilerParams(
            dimension_semantics=("arbitrary", "arbitrary"),
        ),
    )(student_features, teacher_features)

    return _sc_dispatch(teacher_features, idx_rows.reshape(1024))
